# baseline (speedup 1.0000x reference)
import functools

import jax
import jax.numpy as jnp
from jax import lax
from jax.experimental import pallas as pl
from jax.experimental.pallas import tpu as pltpu
from jax.experimental.pallas import tpu_sc as plsc

N_NODES = 10000
N_EDGES = 320000
FEAT = 128
N_GRAPHS = 64

BLK = 128
R_PAD = 10240
NBLK = R_PAD // BLK

NC = 2
NSUB = 16
NW = NC * NSUB
CHUNK = 128
WROWS = 80
E_PAD = NW * WROWS * CHUNK
E_ROWS = E_PAD // CHUNK
TRN = 16
NTR = WROWS // TRN
RPS = R_PAD // NSUB

@functools.cache
def _sc_kernels():
    mesh = plsc.VectorSubcoreMesh(core_axis_name="c", subcore_axis_name="s")

    @functools.partial(
        pl.kernel,
        out_type=jax.ShapeDtypeStruct((NC * R_PAD,), jnp.float32),
        mesh=mesh,
        scratch_types=[
            pltpu.VMEM((WROWS, CHUNK), jnp.int32),
            pltpu.VMEM((CHUNK,), jnp.float32),
            pltpu.VMEM((RPS,), jnp.float32),
            pltpu.VMEM_SHARED((R_PAD,), jnp.float32),
            pltpu.SemaphoreType.DMA,
        ],
    )
    def degree_sc(dst_hbm, out_hbm, dbuf, ones_v, zbuf, dacc, sem):
        c = lax.axis_index("c")
        s = lax.axis_index("s")
        wid = c * NSUB + s

        @pl.loop(0, CHUNK // 16)
        def _(i):
            ones_v[pl.ds(i * 16, 16)] = jnp.ones((16,), jnp.float32)

        @pl.loop(0, RPS // 16)
        def _(i):
            zbuf[pl.ds(i * 16, 16)] = jnp.zeros((16,), jnp.float32)

        pltpu.sync_copy(zbuf, dacc.at[pl.ds(s * RPS, RPS)])
        pltpu.sync_copy(dst_hbm.at[pl.ds(wid * WROWS, WROWS)], dbuf)
        plsc.subcore_barrier()

        @pl.loop(0, WROWS // 4)
        def _(t):
            j = 4 * t
            h0 = pltpu.async_copy(ones_v, dacc.at[dbuf.at[j]], sem, add=True)
            h1 = pltpu.async_copy(ones_v, dacc.at[dbuf.at[j + 1]], sem, add=True)
            h2 = pltpu.async_copy(ones_v, dacc.at[dbuf.at[j + 2]], sem, add=True)
            h3 = pltpu.async_copy(ones_v, dacc.at[dbuf.at[j + 3]], sem, add=True)
            h0.wait()
            h1.wait()
            h2.wait()
            h3.wait()

        plsc.subcore_barrier()
        pltpu.sync_copy(dacc.at[pl.ds(s * RPS, RPS)],
                        out_hbm.at[pl.ds(c * R_PAD + s * RPS, RPS)])

    @functools.partial(
        pl.kernel,
        out_type=jax.ShapeDtypeStruct((NC, R_PAD, FEAT), jnp.float32),
        mesh=mesh,
        scratch_types=[
            pltpu.VMEM((2, TRN, CHUNK), jnp.int32),
            pltpu.VMEM((2, TRN, CHUNK), jnp.int32),
            pltpu.VMEM((CHUNK, FEAT), jnp.float32),
            pltpu.VMEM((CHUNK, FEAT), jnp.float32),
            pltpu.VMEM_SHARED((R_PAD, FEAT), jnp.float32),
            pltpu.SemaphoreType.DMA,
            pltpu.SemaphoreType.DMA,
            pltpu.SemaphoreType.DMA,
        ],
    )
    def propagate_sc(g_hbm, src_hbm, dst_hbm, out_hbm,
                     sbufs, dbufs, rows0, rows1, acc, sem0, sem1, isem):
        c = lax.axis_index("c")
        s = lax.axis_index("s")
        wid = c * NSUB + s

        @pl.loop(0, CHUNK)
        def _(i):
            @pl.loop(0, FEAT // 16)
            def _(j):
                rows0[i, pl.ds(j * 16, 16)] = jnp.zeros((16,), jnp.float32)

        @pl.loop(0, RPS // CHUNK)
        def _(k):
            pltpu.sync_copy(rows0, acc.at[pl.ds(s * RPS + k * CHUNK, CHUNK)])

        base = wid * WROWS
        pltpu.sync_copy(src_hbm.at[pl.ds(base, TRN)], sbufs.at[0])
        pltpu.sync_copy(dst_hbm.at[pl.ds(base, TRN)], dbufs.at[0])
        plsc.subcore_barrier()

        pltpu.async_copy(g_hbm.at[sbufs.at[0].at[0]], rows0, sem0)

        @pl.loop(0, NTR)
        def _(t):
            r = lax.rem(t, 2)
            rn = lax.rem(t + 1, 2)
            tn = jnp.minimum(t + 1, NTR - 1)
            hs = pltpu.async_copy(src_hbm.at[pl.ds(base + tn * TRN, TRN)],
                                  sbufs.at[rn], isem)
            hd = pltpu.async_copy(dst_hbm.at[pl.ds(base + tn * TRN, TRN)],
                                  dbufs.at[rn], isem)
            sb = sbufs.at[r]
            db = dbufs.at[r]

            @pl.loop(0, TRN // 2)
            def _(u):
                j0 = 2 * u
                j1 = j0 + 1
                jn = jnp.minimum(j0 + 2, TRN - 1)
                pltpu.async_copy(g_hbm.at[sb.at[j1]], rows1, sem1)
                pltpu.make_async_copy(g_hbm.at[sb.at[j0]], rows0, sem0).wait()
                pltpu.sync_copy(rows0, acc.at[db.at[j0]], add=True)
                pltpu.async_copy(g_hbm.at[sb.at[jn]], rows0, sem0)
                pltpu.make_async_copy(g_hbm.at[sb.at[j1]], rows1, sem1).wait()
                pltpu.sync_copy(rows1, acc.at[db.at[j1]], add=True)

            pltpu.make_async_copy(g_hbm.at[sb.at[TRN - 1]], rows0, sem0).wait()
            hs.wait()
            hd.wait()
            pltpu.async_copy(g_hbm.at[sbufs.at[rn].at[0]], rows0, sem0)

        pltpu.make_async_copy(g_hbm.at[sbufs.at[0].at[0]], rows0, sem0).wait()

        plsc.subcore_barrier()
        pltpu.sync_copy(acc.at[pl.ds(s * RPS, RPS)],
                        out_hbm.at[c, pl.ds(s * RPS, RPS)])

    return degree_sc, propagate_sc


def _c1_body(x_ref, d0_ref, d1_ref, w_ref, g_ref):
    dv = lax.rsqrt(d0_ref[...] + d1_ref[...] + 1.0)
    g_ref[...] = jnp.dot(x_ref[...], w_ref[...],
                         preferred_element_type=jnp.float32) * dv


def _cmid_body(a0_ref, a1_ref, gp_ref, d0_ref, d1_ref, b_ref, w_ref, oh_ref,
               g_ref, pool_ref):
    dv = lax.rsqrt(d0_ref[...] + d1_ref[...] + 1.0)
    act = jnp.maximum((a0_ref[...] + a1_ref[...] + gp_ref[...]) * dv
                      + b_ref[...], 0.0)

    @pl.when(pl.program_id(0) == 0)
    def _():
        pool_ref[...] = jnp.zeros_like(pool_ref)

    pool_ref[...] += lax.dot_general(
        oh_ref[...], act, (((0,), (0,)), ((), ())),
        preferred_element_type=jnp.float32)
    g_ref[...] = jnp.dot(act, w_ref[...],
                         preferred_element_type=jnp.float32) * dv


def _cfin_body(a0_ref, a1_ref, gp_ref, d0_ref, d1_ref, b_ref, oh_ref,
               pool_ref):
    dv = lax.rsqrt(d0_ref[...] + d1_ref[...] + 1.0)
    act = jnp.maximum((a0_ref[...] + a1_ref[...] + gp_ref[...]) * dv
                      + b_ref[...], 0.0)

    @pl.when(pl.program_id(0) == 0)
    def _():
        pool_ref[...] = jnp.zeros_like(pool_ref)

    pool_ref[...] += lax.dot_general(
        oh_ref[...], act, (((0,), (0,)), ((), ())),
        preferred_element_type=jnp.float32)


_row_spec = pl.BlockSpec((BLK, FEAT), lambda m: (m, 0))
_col_spec = pl.BlockSpec((BLK, 1), lambda m: (m, 0))
_w_spec = pl.BlockSpec((FEAT, FEAT), lambda m: (0, 0))
_b_spec = pl.BlockSpec((1, FEAT), lambda m: (0, 0))
_oh_spec = pl.BlockSpec((BLK, N_GRAPHS), lambda m: (m, 0))
_pool_spec = pl.BlockSpec((N_GRAPHS, FEAT), lambda m: (0, 0))


def _c1_tc(x_pad, d0, d1, W):
    return pl.pallas_call(
        _c1_body,
        grid=(NBLK,),
        in_specs=[_row_spec, _col_spec, _col_spec, _w_spec],
        out_specs=_row_spec,
        out_shape=jax.ShapeDtypeStruct((R_PAD, FEAT), jnp.float32),
    )(x_pad, d0, d1, W)


def _cmid_tc(a0, a1, gp, d0, d1, b, W, oh):
    return pl.pallas_call(
        _cmid_body,
        grid=(NBLK,),
        in_specs=[_row_spec, _row_spec, _row_spec, _col_spec, _col_spec,
                  _b_spec, _w_spec, _oh_spec],
        out_specs=[_row_spec, _pool_spec],
        out_shape=[jax.ShapeDtypeStruct((R_PAD, FEAT), jnp.float32),
                   jax.ShapeDtypeStruct((N_GRAPHS, FEAT), jnp.float32)],
    )(a0, a1, gp, d0, d1, b, W, oh)


def _cfin_tc(a0, a1, gp, d0, d1, b, oh):
    return pl.pallas_call(
        _cfin_body,
        grid=(NBLK,),
        in_specs=[_row_spec, _row_spec, _row_spec, _col_spec, _col_spec,
                  _b_spec, _oh_spec],
        out_specs=_pool_spec,
        out_shape=jax.ShapeDtypeStruct((N_GRAPHS, FEAT), jnp.float32),
    )(a0, a1, gp, d0, d1, b, oh)


def kernel(x, edge_index, batch, W1, b1, W2, b2, W3, b3):
    pad_n = E_PAD - N_EDGES
    src = jnp.concatenate(
        [edge_index[0].astype(jnp.int32),
         jnp.zeros((pad_n,), jnp.int32)]).reshape(E_ROWS, CHUNK)
    dst = jnp.concatenate(
        [edge_index[1].astype(jnp.int32),
         N_NODES + (jnp.arange(pad_n, dtype=jnp.int32)
                    % (R_PAD - N_NODES))]).reshape(E_ROWS, CHUNK)

    x_pad = jnp.pad(x, ((0, R_PAD - N_NODES), (0, 0)))
    oh = (batch[:, None] == jnp.arange(N_GRAPHS, dtype=batch.dtype)[None, :])
    oh = jnp.pad(oh.astype(jnp.float32), ((0, R_PAD - N_NODES), (0, 0)))
    b1r = b1.reshape(1, FEAT)
    b2r = b2.reshape(1, FEAT)
    b3r = b3.reshape(1, FEAT)

    degree_sc, propagate_sc = _sc_kernels()
    deg = degree_sc(dst).reshape(NC, R_PAD)
    d0 = deg[0].reshape(R_PAD, 1)
    d1 = deg[1].reshape(R_PAD, 1)

    g1 = _c1_tc(x_pad, d0, d1, W1)
    a1 = propagate_sc(g1, src, dst)
    g2, pool1 = _cmid_tc(a1[0], a1[1], g1, d0, d1, b1r, W2, oh)
    a2 = propagate_sc(g2, src, dst)
    g3, pool2 = _cmid_tc(a2[0], a2[1], g2, d0, d1, b2r, W3, oh)
    a3 = propagate_sc(g3, src, dst)
    pool3 = _cfin_tc(a3[0], a3[1], g3, d0, d1, b3r, oh)

    return jnp.concatenate([pool1, pool2, pool3], axis=1)


# speedup vs baseline: 2.6570x; 2.6570x over previous
import functools

import jax
import jax.numpy as jnp
from jax import lax
from jax.experimental import pallas as pl
from jax.experimental.pallas import tpu as pltpu
from jax.experimental.pallas import tpu_sc as plsc

N_NODES = 10000
N_EDGES = 320000
FEAT = 128
N_GRAPHS = 64

BLK = 128
R_PAD = 10240
NBLK = R_PAD // BLK

NC = 2
NSUB = 16
NW = NC * NSUB
CHUNK = 128
WROWS = 80
E_PAD = NW * WROWS * CHUNK
E_ROWS = E_PAD // CHUNK
TRN = 16
NTR = WROWS // TRN
RPS = R_PAD // NSUB

@functools.cache
def _sc_kernels():
    mesh = plsc.VectorSubcoreMesh(core_axis_name="c", subcore_axis_name="s")

    @functools.partial(
        pl.kernel,
        out_type=jax.ShapeDtypeStruct((NC * R_PAD,), jnp.float32),
        mesh=mesh,
        scratch_types=[
            pltpu.VMEM((WROWS, CHUNK), jnp.int32),
            pltpu.VMEM((CHUNK,), jnp.float32),
            pltpu.VMEM((RPS,), jnp.float32),
            pltpu.VMEM_SHARED((R_PAD,), jnp.float32),
            pltpu.SemaphoreType.DMA,
        ],
    )
    def degree_sc(dst_hbm, out_hbm, dbuf, ones_v, zbuf, dacc, sem):
        c = lax.axis_index("c")
        s = lax.axis_index("s")
        wid = c * NSUB + s

        @pl.loop(0, CHUNK // 16)
        def _(i):
            ones_v[pl.ds(i * 16, 16)] = jnp.ones((16,), jnp.float32)

        @pl.loop(0, RPS // 16)
        def _(i):
            zbuf[pl.ds(i * 16, 16)] = jnp.zeros((16,), jnp.float32)

        pltpu.sync_copy(zbuf, dacc.at[pl.ds(s * RPS, RPS)])
        pltpu.sync_copy(dst_hbm.at[pl.ds(wid * WROWS, WROWS)], dbuf)
        plsc.subcore_barrier()

        @pl.loop(0, WROWS // 4)
        def _(t):
            j = 4 * t
            h0 = pltpu.async_copy(ones_v, dacc.at[dbuf.at[j]], sem, add=True)
            h1 = pltpu.async_copy(ones_v, dacc.at[dbuf.at[j + 1]], sem, add=True)
            h2 = pltpu.async_copy(ones_v, dacc.at[dbuf.at[j + 2]], sem, add=True)
            h3 = pltpu.async_copy(ones_v, dacc.at[dbuf.at[j + 3]], sem, add=True)
            h0.wait()
            h1.wait()
            h2.wait()
            h3.wait()

        plsc.subcore_barrier()
        pltpu.sync_copy(dacc.at[pl.ds(s * RPS, RPS)],
                        out_hbm.at[pl.ds(c * R_PAD + s * RPS, RPS)])

    @functools.partial(
        pl.kernel,
        out_type=jax.ShapeDtypeStruct((NC, R_PAD, FEAT), jnp.float32),
        mesh=mesh,
        scratch_types=[
            pltpu.VMEM((2, TRN, CHUNK), jnp.int32),
            pltpu.VMEM((2, TRN, CHUNK), jnp.int32),
            pltpu.VMEM((CHUNK, FEAT), jnp.float32),
            pltpu.VMEM((CHUNK, FEAT), jnp.float32),
            pltpu.VMEM_SHARED((R_PAD, FEAT), jnp.float32),
            pltpu.SemaphoreType.DMA,
            pltpu.SemaphoreType.DMA,
            pltpu.SemaphoreType.DMA,
        ],
    )
    def propagate_sc(g_hbm, src_hbm, dst_hbm, out_hbm,
                     sbufs, dbufs, rows0, rows1, acc, sem0, sem1, isem):
        c = lax.axis_index("c")
        s = lax.axis_index("s")
        wid = c * NSUB + s

        @pl.loop(0, CHUNK)
        def _(i):
            @pl.loop(0, FEAT // 16)
            def _(j):
                rows0[i, pl.ds(j * 16, 16)] = jnp.zeros((16,), jnp.float32)

        @pl.loop(0, RPS // CHUNK)
        def _(k):
            pltpu.sync_copy(rows0, acc.at[pl.ds(s * RPS + k * CHUNK, CHUNK)])

        base = wid * WROWS
        pltpu.sync_copy(src_hbm.at[pl.ds(base, TRN)], sbufs.at[0])
        pltpu.sync_copy(dst_hbm.at[pl.ds(base, TRN)], dbufs.at[0])
        plsc.subcore_barrier()

        pltpu.async_copy(g_hbm.at[sbufs.at[0].at[0]], rows0, sem0)

        @pl.loop(0, NTR)
        def _(t):
            r = lax.rem(t, 2)
            rn = lax.rem(t + 1, 2)
            tn = jnp.minimum(t + 1, NTR - 1)
            hs = pltpu.async_copy(src_hbm.at[pl.ds(base + tn * TRN, TRN)],
                                  sbufs.at[rn], isem)
            hd = pltpu.async_copy(dst_hbm.at[pl.ds(base + tn * TRN, TRN)],
                                  dbufs.at[rn], isem)
            sb = sbufs.at[r]
            db = dbufs.at[r]

            @pl.loop(0, TRN // 2)
            def _(u):
                j0 = 2 * u
                j1 = j0 + 1
                jn = jnp.minimum(j0 + 2, TRN - 1)
                pltpu.async_copy(g_hbm.at[sb.at[j1]], rows1, sem1)
                pltpu.make_async_copy(g_hbm.at[sb.at[j0]], rows0, sem0).wait()
                pltpu.sync_copy(rows0, acc.at[db.at[j0]], add=True)
                pltpu.async_copy(g_hbm.at[sb.at[jn]], rows0, sem0)
                pltpu.make_async_copy(g_hbm.at[sb.at[j1]], rows1, sem1).wait()
                pltpu.sync_copy(rows1, acc.at[db.at[j1]], add=True)

            pltpu.make_async_copy(g_hbm.at[sb.at[TRN - 1]], rows0, sem0).wait()
            hs.wait()
            hd.wait()
            pltpu.async_copy(g_hbm.at[sbufs.at[rn].at[0]], rows0, sem0)

        pltpu.make_async_copy(g_hbm.at[sbufs.at[0].at[0]], rows0, sem0).wait()

        plsc.subcore_barrier()
        pltpu.sync_copy(acc.at[pl.ds(s * RPS, RPS)],
                        out_hbm.at[c, pl.ds(s * RPS, RPS)])

    return degree_sc, propagate_sc


def _c1_body(x_ref, d0_ref, d1_ref, w_ref, g_ref):
    dv = lax.rsqrt(d0_ref[...] + d1_ref[...] + 1.0)
    g_ref[...] = jnp.dot(x_ref[...], w_ref[...],
                         preferred_element_type=jnp.float32) * dv


def _cmid_body(a0_ref, a1_ref, gp_ref, d0_ref, d1_ref, b_ref, w_ref, oh_ref,
               g_ref, pool_ref):
    dv = lax.rsqrt(d0_ref[...] + d1_ref[...] + 1.0)
    act = jnp.maximum((a0_ref[...] + a1_ref[...] + gp_ref[...]) * dv
                      + b_ref[...], 0.0)

    @pl.when(pl.program_id(0) == 0)
    def _():
        pool_ref[...] = jnp.zeros_like(pool_ref)

    pool_ref[...] += lax.dot_general(
        oh_ref[...], act, (((0,), (0,)), ((), ())),
        preferred_element_type=jnp.float32)
    g_ref[...] = jnp.dot(act, w_ref[...],
                         preferred_element_type=jnp.float32) * dv


def _cfin_body(a0_ref, a1_ref, gp_ref, d0_ref, d1_ref, b_ref, oh_ref,
               pool_ref):
    dv = lax.rsqrt(d0_ref[...] + d1_ref[...] + 1.0)
    act = jnp.maximum((a0_ref[...] + a1_ref[...] + gp_ref[...]) * dv
                      + b_ref[...], 0.0)

    @pl.when(pl.program_id(0) == 0)
    def _():
        pool_ref[...] = jnp.zeros_like(pool_ref)

    pool_ref[...] += lax.dot_general(
        oh_ref[...], act, (((0,), (0,)), ((), ())),
        preferred_element_type=jnp.float32)


_row_spec = pl.BlockSpec((BLK, FEAT), lambda m: (m, 0))
_col_spec = pl.BlockSpec((BLK, 1), lambda m: (m, 0))
_w_spec = pl.BlockSpec((FEAT, FEAT), lambda m: (0, 0))
_b_spec = pl.BlockSpec((1, FEAT), lambda m: (0, 0))
_oh_spec = pl.BlockSpec((BLK, N_GRAPHS), lambda m: (m, 0))
_pool_spec = pl.BlockSpec((N_GRAPHS, FEAT), lambda m: (0, 0))


def _c1_tc(x_pad, d0, d1, W):
    return pl.pallas_call(
        _c1_body,
        grid=(NBLK,),
        in_specs=[_row_spec, _col_spec, _col_spec, _w_spec],
        out_specs=_row_spec,
        out_shape=jax.ShapeDtypeStruct((R_PAD, FEAT), jnp.float32),
    )(x_pad, d0, d1, W)


def _cmid_tc(a0, a1, gp, d0, d1, b, W, oh):
    return pl.pallas_call(
        _cmid_body,
        grid=(NBLK,),
        in_specs=[_row_spec, _row_spec, _row_spec, _col_spec, _col_spec,
                  _b_spec, _w_spec, _oh_spec],
        out_specs=[_row_spec, _pool_spec],
        out_shape=[jax.ShapeDtypeStruct((R_PAD, FEAT), jnp.float32),
                   jax.ShapeDtypeStruct((N_GRAPHS, FEAT), jnp.float32)],
    )(a0, a1, gp, d0, d1, b, W, oh)


def _cfin_tc(a0, a1, gp, d0, d1, b, oh):
    return pl.pallas_call(
        _cfin_body,
        grid=(NBLK,),
        in_specs=[_row_spec, _row_spec, _row_spec, _col_spec, _col_spec,
                  _b_spec, _oh_spec],
        out_specs=_pool_spec,
        out_shape=jax.ShapeDtypeStruct((N_GRAPHS, FEAT), jnp.float32),
    )(a0, a1, gp, d0, d1, b, oh)


def kernel(x, edge_index, batch, W1, b1, W2, b2, W3, b3):
    pad_n = E_PAD - N_EDGES
    src = jnp.concatenate(
        [edge_index[0].astype(jnp.int32),
         (jnp.arange(pad_n, dtype=jnp.int32) * 997) % N_NODES]
    ).reshape(E_ROWS, CHUNK)
    dst = jnp.concatenate(
        [edge_index[1].astype(jnp.int32),
         N_NODES + (jnp.arange(pad_n, dtype=jnp.int32)
                    % (R_PAD - N_NODES))]).reshape(E_ROWS, CHUNK)

    x_pad = jnp.pad(x, ((0, R_PAD - N_NODES), (0, 0)))
    oh = (batch[:, None] == jnp.arange(N_GRAPHS, dtype=batch.dtype)[None, :])
    oh = jnp.pad(oh.astype(jnp.float32), ((0, R_PAD - N_NODES), (0, 0)))
    b1r = b1.reshape(1, FEAT)
    b2r = b2.reshape(1, FEAT)
    b3r = b3.reshape(1, FEAT)

    degree_sc, propagate_sc = _sc_kernels()
    deg = degree_sc(dst).reshape(NC, R_PAD)
    d0 = deg[0].reshape(R_PAD, 1)
    d1 = deg[1].reshape(R_PAD, 1)

    g1 = _c1_tc(x_pad, d0, d1, W1)
    a1 = propagate_sc(g1, src, dst)
    g2, pool1 = _cmid_tc(a1[0], a1[1], g1, d0, d1, b1r, W2, oh)
    a2 = propagate_sc(g2, src, dst)
    g3, pool2 = _cmid_tc(a2[0], a2[1], g2, d0, d1, b2r, W3, oh)
    a3 = propagate_sc(g3, src, dst)
    pool3 = _cfin_tc(a3[0], a3[1], g3, d0, d1, b3r, oh)

    return jnp.concatenate([pool1, pool2, pool3], axis=1)


# speedup vs baseline: 2.9673x; 1.1168x over previous
import functools

import jax
import jax.numpy as jnp
from jax import lax
from jax.experimental import pallas as pl
from jax.experimental.pallas import tpu as pltpu
from jax.experimental.pallas import tpu_sc as plsc

N_NODES = 10000
N_EDGES = 320000
FEAT = 128
N_GRAPHS = 64

BLK = 128
BLKH = 256

R_PAD = 10240
NBLK = R_PAD // BLK
NBLKH = R_PAD // BLKH

NC = 2
NSUB = 16
NW = NC * NSUB
CHUNK = 128
WROWS = 80
E_PAD = NW * WROWS * CHUNK
E_ROWS = E_PAD // CHUNK
TRN = 16
NTR = WROWS // TRN
RPS = R_PAD // NSUB

@functools.cache
def _sc_kernels():
    mesh = plsc.VectorSubcoreMesh(core_axis_name="c", subcore_axis_name="s")

    @functools.partial(
        pl.kernel,
        out_type=jax.ShapeDtypeStruct((NC * R_PAD,), jnp.float32),
        mesh=mesh,
        scratch_types=[
            pltpu.VMEM((WROWS, CHUNK), jnp.int32),
            pltpu.VMEM((CHUNK,), jnp.float32),
            pltpu.VMEM((RPS,), jnp.float32),
            pltpu.VMEM_SHARED((R_PAD,), jnp.float32),
            pltpu.SemaphoreType.DMA,
        ],
    )
    def degree_sc(dst_hbm, out_hbm, dbuf, ones_v, zbuf, dacc, sem):
        c = lax.axis_index("c")
        s = lax.axis_index("s")
        wid = c * NSUB + s

        @pl.loop(0, CHUNK // 16)
        def _(i):
            ones_v[pl.ds(i * 16, 16)] = jnp.ones((16,), jnp.float32)

        @pl.loop(0, RPS // 16)
        def _(i):
            zbuf[pl.ds(i * 16, 16)] = jnp.zeros((16,), jnp.float32)

        pltpu.sync_copy(zbuf, dacc.at[pl.ds(s * RPS, RPS)])
        pltpu.sync_copy(dst_hbm.at[pl.ds(wid * WROWS, WROWS)], dbuf)
        plsc.subcore_barrier()

        @pl.loop(0, WROWS // 4)
        def _(t):
            j = 4 * t
            h0 = pltpu.async_copy(ones_v, dacc.at[dbuf.at[j]], sem, add=True)
            h1 = pltpu.async_copy(ones_v, dacc.at[dbuf.at[j + 1]], sem, add=True)
            h2 = pltpu.async_copy(ones_v, dacc.at[dbuf.at[j + 2]], sem, add=True)
            h3 = pltpu.async_copy(ones_v, dacc.at[dbuf.at[j + 3]], sem, add=True)
            h0.wait()
            h1.wait()
            h2.wait()
            h3.wait()

        plsc.subcore_barrier()
        pltpu.sync_copy(dacc.at[pl.ds(s * RPS, RPS)],
                        out_hbm.at[pl.ds(c * R_PAD + s * RPS, RPS)])

    @functools.partial(
        pl.kernel,
        out_type=jax.ShapeDtypeStruct((NC, R_PAD, FEAT), jnp.float32),
        mesh=mesh,
        scratch_types=[
            pltpu.VMEM((2, TRN, CHUNK), jnp.int32),
            pltpu.VMEM((2, TRN, CHUNK), jnp.int32),
            pltpu.VMEM((CHUNK, FEAT), jnp.float32),
            pltpu.VMEM((CHUNK, FEAT), jnp.float32),
            pltpu.VMEM_SHARED((R_PAD, FEAT), jnp.float32),
            pltpu.SemaphoreType.DMA,
            pltpu.SemaphoreType.DMA,
            pltpu.SemaphoreType.DMA,
        ],
    )
    def propagate_sc(g_hbm, src_hbm, dst_hbm, out_hbm,
                     sbufs, dbufs, rows0, rows1, acc, sem0, sem1, isem):
        c = lax.axis_index("c")
        s = lax.axis_index("s")
        wid = c * NSUB + s

        @pl.loop(0, CHUNK)
        def _(i):
            @pl.loop(0, FEAT // 16)
            def _(j):
                rows0[i, pl.ds(j * 16, 16)] = jnp.zeros((16,), jnp.float32)

        @pl.loop(0, RPS // CHUNK)
        def _(k):
            pltpu.sync_copy(rows0, acc.at[pl.ds(s * RPS + k * CHUNK, CHUNK)])

        base = wid * WROWS
        pltpu.sync_copy(src_hbm.at[pl.ds(base, TRN)], sbufs.at[0])
        pltpu.sync_copy(dst_hbm.at[pl.ds(base, TRN)], dbufs.at[0])
        plsc.subcore_barrier()

        pltpu.async_copy(g_hbm.at[sbufs.at[0].at[0]], rows0, sem0)

        @pl.loop(0, NTR)
        def _(t):
            r = lax.rem(t, 2)
            rn = lax.rem(t + 1, 2)
            tn = jnp.minimum(t + 1, NTR - 1)
            hs = pltpu.async_copy(src_hbm.at[pl.ds(base + tn * TRN, TRN)],
                                  sbufs.at[rn], isem)
            hd = pltpu.async_copy(dst_hbm.at[pl.ds(base + tn * TRN, TRN)],
                                  dbufs.at[rn], isem)
            sb = sbufs.at[r]
            db = dbufs.at[r]

            @pl.loop(0, TRN // 2)
            def _(u):
                j0 = 2 * u
                j1 = j0 + 1
                jn = jnp.minimum(j0 + 2, TRN - 1)
                pltpu.async_copy(g_hbm.at[sb.at[j1]], rows1, sem1)
                pltpu.make_async_copy(g_hbm.at[sb.at[j0]], rows0, sem0).wait()
                pltpu.sync_copy(rows0, acc.at[db.at[j0]], add=True)
                pltpu.async_copy(g_hbm.at[sb.at[jn]], rows0, sem0)
                pltpu.make_async_copy(g_hbm.at[sb.at[j1]], rows1, sem1).wait()
                pltpu.sync_copy(rows1, acc.at[db.at[j1]], add=True)

            pltpu.make_async_copy(g_hbm.at[sb.at[TRN - 1]], rows0, sem0).wait()
            hs.wait()
            hd.wait()
            pltpu.async_copy(g_hbm.at[sbufs.at[rn].at[0]], rows0, sem0)

        pltpu.make_async_copy(g_hbm.at[sbufs.at[0].at[0]], rows0, sem0).wait()

        plsc.subcore_barrier()
        pltpu.sync_copy(acc.at[pl.ds(s * RPS, RPS)],
                        out_hbm.at[c, pl.ds(s * RPS, RPS)])

    return degree_sc, propagate_sc


def _prep_body(dp_ref, dcol_ref):
    dv = lax.rsqrt(dp_ref[0, 0, 0, :] + dp_ref[1, 0, 0, :] + 1.0).reshape(1, BLK)
    rr = lax.broadcasted_iota(jnp.int32, (BLK, BLK), 0)
    cc = lax.broadcasted_iota(jnp.int32, (BLK, BLK), 1)
    diag = jnp.where(rr == cc, jnp.broadcast_to(dv, (BLK, BLK)), 0.0)
    dcol_ref[...] = jnp.dot(diag, jnp.ones((BLK, BLK), jnp.float32),
                            preferred_element_type=jnp.float32)


def _c1_body(x_ref, dcol_ref, w_ref, g_ref):
    g_ref[...] = jnp.dot(x_ref[...], w_ref[...],
                         preferred_element_type=jnp.float32) * dcol_ref[...]


def _cmid_body(a0_ref, a1_ref, gp_ref, dcol_ref, b_ref, w_ref, oh_ref,
               g_ref, pool_ref):
    dcol = dcol_ref[...]
    act = jnp.maximum((a0_ref[...] + a1_ref[...] + gp_ref[...]) * dcol
                      + b_ref[...], 0.0)

    @pl.when(pl.program_id(0) == 0)
    def _():
        pool_ref[...] = jnp.zeros_like(pool_ref)

    pool_ref[...] += lax.dot_general(
        oh_ref[...], act, (((0,), (0,)), ((), ())),
        preferred_element_type=jnp.float32)
    g_ref[...] = jnp.dot(act, w_ref[...],
                         preferred_element_type=jnp.float32) * dcol


def _cfin_body(a0_ref, a1_ref, gp_ref, dcol_ref, b_ref, oh_ref, pool_ref):
    act = jnp.maximum((a0_ref[...] + a1_ref[...] + gp_ref[...]) * dcol_ref[...]
                      + b_ref[...], 0.0)

    @pl.when(pl.program_id(0) == 0)
    def _():
        pool_ref[...] = jnp.zeros_like(pool_ref)

    pool_ref[...] += lax.dot_general(
        oh_ref[...], act, (((0,), (0,)), ((), ())),
        preferred_element_type=jnp.float32)


_row_spec = pl.BlockSpec((BLKH, FEAT), lambda m: (m, 0))
_w_spec = pl.BlockSpec((FEAT, FEAT), lambda m: (0, 0))
_b_spec = pl.BlockSpec((1, FEAT), lambda m: (0, 0))
_oh_spec = pl.BlockSpec((BLKH, N_GRAPHS), lambda m: (m, 0))
_pool_spec = pl.BlockSpec((N_GRAPHS, FEAT), lambda m: (0, 0))


def _prep_tc(dp):
    return pl.pallas_call(
        _prep_body,
        grid=(NBLK,),
        in_specs=[pl.BlockSpec((2, 1, 1, BLK), lambda m: (0, m, 0, 0))],
        out_specs=pl.BlockSpec((BLK, BLK), lambda m: (m, 0)),
        out_shape=jax.ShapeDtypeStruct((R_PAD, FEAT), jnp.float32),
    )(dp)


def _c1_tc(x_pad, dcol, W):
    return pl.pallas_call(
        _c1_body,
        grid=(NBLKH,),
        in_specs=[_row_spec, _row_spec, _w_spec],
        out_specs=_row_spec,
        out_shape=jax.ShapeDtypeStruct((R_PAD, FEAT), jnp.float32),
    )(x_pad, dcol, W)


def _cmid_tc(a0, a1, gp, dcol, b, W, oh):
    return pl.pallas_call(
        _cmid_body,
        grid=(NBLKH,),
        in_specs=[_row_spec, _row_spec, _row_spec, _row_spec,
                  _b_spec, _w_spec, _oh_spec],
        out_specs=[_row_spec, _pool_spec],
        out_shape=[jax.ShapeDtypeStruct((R_PAD, FEAT), jnp.float32),
                   jax.ShapeDtypeStruct((N_GRAPHS, FEAT), jnp.float32)],
    )(a0, a1, gp, dcol, b, W, oh)


def _cfin_tc(a0, a1, gp, dcol, b, oh):
    return pl.pallas_call(
        _cfin_body,
        grid=(NBLKH,),
        in_specs=[_row_spec, _row_spec, _row_spec, _row_spec,
                  _b_spec, _oh_spec],
        out_specs=_pool_spec,
        out_shape=jax.ShapeDtypeStruct((N_GRAPHS, FEAT), jnp.float32),
    )(a0, a1, gp, dcol, b, oh)


def kernel(x, edge_index, batch, W1, b1, W2, b2, W3, b3):
    pad_n = E_PAD - N_EDGES
    src = jnp.concatenate(
        [edge_index[0].astype(jnp.int32),
         (jnp.arange(pad_n, dtype=jnp.int32) * 997) % N_NODES]
    ).reshape(E_ROWS, CHUNK)
    dst = jnp.concatenate(
        [edge_index[1].astype(jnp.int32),
         N_NODES + (jnp.arange(pad_n, dtype=jnp.int32)
                    % (R_PAD - N_NODES))]).reshape(E_ROWS, CHUNK)

    x_pad = jnp.pad(x, ((0, R_PAD - N_NODES), (0, 0)))
    oh = (batch[:, None] == jnp.arange(N_GRAPHS, dtype=batch.dtype)[None, :])
    oh = jnp.pad(oh.astype(jnp.float32), ((0, R_PAD - N_NODES), (0, 0)))
    b1r = b1.reshape(1, FEAT)
    b2r = b2.reshape(1, FEAT)
    b3r = b3.reshape(1, FEAT)

    degree_sc, propagate_sc = _sc_kernels()
    dp = degree_sc(dst).reshape(NC, NBLK, 1, BLK)
    dcol = _prep_tc(dp)

    g1 = _c1_tc(x_pad, dcol, W1)
    a1 = propagate_sc(g1, src, dst)
    g2, pool1 = _cmid_tc(a1[0], a1[1], g1, dcol, b1r, W2, oh)
    a2 = propagate_sc(g2, src, dst)
    g3, pool2 = _cmid_tc(a2[0], a2[1], g2, dcol, b2r, W3, oh)
    a3 = propagate_sc(g3, src, dst)
    pool3 = _cfin_tc(a3[0], a3[1], g3, dcol, b3r, oh)

    return jnp.concatenate([pool1, pool2, pool3], axis=1)


# speedup vs baseline: 3.4638x; 1.1673x over previous
import functools

import jax
import jax.numpy as jnp
from jax import lax
from jax.experimental import pallas as pl
from jax.experimental.pallas import tpu as pltpu
from jax.experimental.pallas import tpu_sc as plsc

N_NODES = 10000
N_EDGES = 320000
FEAT = 128
N_GRAPHS = 64

BLK = 128
BLKH = 256

R_PAD = 10240
NBLK = R_PAD // BLK
NBLKH = R_PAD // BLKH

NC = 2
NSUB = 16
NW = NC * NSUB
CHUNK = 80
WROWS = 128
E_PAD = NW * WROWS * CHUNK
E_ROWS = E_PAD // CHUNK
TRN = 32
NTRN = WROWS // TRN
RPS = R_PAD // NSUB

@functools.cache
def _sc_kernels():
    mesh = plsc.VectorSubcoreMesh(core_axis_name="c", subcore_axis_name="s")

    @functools.partial(
        pl.kernel,
        out_type=jax.ShapeDtypeStruct((NC * R_PAD,), jnp.float32),
        mesh=mesh,
        scratch_types=[
            pltpu.VMEM((WROWS, CHUNK), jnp.int32),
            pltpu.VMEM((CHUNK,), jnp.float32),
            pltpu.VMEM((RPS,), jnp.float32),
            pltpu.VMEM_SHARED((R_PAD,), jnp.float32),
            pltpu.SemaphoreType.DMA,
        ],
    )
    def degree_sc(dst_hbm, out_hbm, dbuf, ones_v, zbuf, dacc, sem):
        c = lax.axis_index("c")
        s = lax.axis_index("s")
        wid = c * NSUB + s

        @pl.loop(0, CHUNK // 16)
        def _(i):
            ones_v[pl.ds(i * 16, 16)] = jnp.ones((16,), jnp.float32)

        @pl.loop(0, RPS // 16)
        def _(i):
            zbuf[pl.ds(i * 16, 16)] = jnp.zeros((16,), jnp.float32)

        pltpu.sync_copy(zbuf, dacc.at[pl.ds(s * RPS, RPS)])
        pltpu.sync_copy(dst_hbm.at[pl.ds(wid * WROWS, WROWS)], dbuf)
        plsc.subcore_barrier()

        @pl.loop(0, WROWS // 4)
        def _(t):
            j = 4 * t
            h0 = pltpu.async_copy(ones_v, dacc.at[dbuf.at[j]], sem, add=True)
            h1 = pltpu.async_copy(ones_v, dacc.at[dbuf.at[j + 1]], sem, add=True)
            h2 = pltpu.async_copy(ones_v, dacc.at[dbuf.at[j + 2]], sem, add=True)
            h3 = pltpu.async_copy(ones_v, dacc.at[dbuf.at[j + 3]], sem, add=True)
            h0.wait()
            h1.wait()
            h2.wait()
            h3.wait()

        plsc.subcore_barrier()
        pltpu.sync_copy(dacc.at[pl.ds(s * RPS, RPS)],
                        out_hbm.at[pl.ds(c * R_PAD + s * RPS, RPS)])

    @functools.partial(
        pl.kernel,
        out_type=jax.ShapeDtypeStruct((NC, R_PAD, FEAT), jnp.float32),
        mesh=mesh,
        scratch_types=[
            pltpu.VMEM((2, 2, TRN, CHUNK), jnp.int32),
            pltpu.VMEM((3 * CHUNK, FEAT), jnp.float32),
            pltpu.VMEM_SHARED((R_PAD, FEAT), jnp.float32),
            pltpu.SemaphoreType.DMA,
            pltpu.SemaphoreType.DMA,
            pltpu.SemaphoreType.DMA,
            pltpu.SemaphoreType.DMA,
        ],
    )
    def propagate_sc(g_hbm, e2_hbm, out_hbm,
                     ibuf, rowsb, acc, sem0, sem1, sem2, isem):
        c = lax.axis_index("c")
        s = lax.axis_index("s")
        wid = c * NSUB + s
        rows = [rowsb.at[pl.ds(k * CHUNK, CHUNK)] for k in range(3)]
        sems = [sem0, sem1, sem2]

        @pl.loop(0, 3 * CHUNK)
        def _(i):
            @pl.loop(0, FEAT // 16)
            def _(j):
                rowsb[i, pl.ds(j * 16, 16)] = jnp.zeros((16,), jnp.float32)

        @pl.loop(0, RPS // CHUNK)
        def _(k):
            pltpu.sync_copy(rows[0], acc.at[pl.ds(s * RPS + k * CHUNK, CHUNK)])

        base = wid * WROWS
        pltpu.sync_copy(e2_hbm.at[:, pl.ds(base, TRN)], ibuf.at[0])
        plsc.subcore_barrier()

        for T in range(NTRN):
            r = T % 2
            rn = (T + 1) % 2
            if T < NTRN - 1:
                hpre = pltpu.async_copy(
                    e2_hbm.at[:, pl.ds(base + (T + 1) * TRN, TRN)],
                    ibuf.at[rn], isem)
            sb = ibuf.at[r].at[0]
            db = ibuf.at[r].at[1]
            for u in range(3):
                pltpu.async_copy(g_hbm.at[sb.at[u]], rows[u], sems[u])

            @pl.loop(0, TRN // 3 - 1)
            def _(t):
                for lane in range(3):
                    j = 3 * t + lane
                    pltpu.make_async_copy(
                        g_hbm.at[sb.at[j]], rows[lane], sems[lane]).wait()
                    pltpu.sync_copy(rows[lane], acc.at[db.at[j]], add=True)
                    pltpu.async_copy(
                        g_hbm.at[sb.at[j + 3]], rows[lane], sems[lane])

            for lane, j in ((0, TRN - 5), (1, TRN - 4)):
                pltpu.make_async_copy(
                    g_hbm.at[sb.at[j]], rows[lane], sems[lane]).wait()
                pltpu.sync_copy(rows[lane], acc.at[db.at[j]], add=True)
                pltpu.async_copy(
                    g_hbm.at[sb.at[j + 3]], rows[lane], sems[lane])
            for lane, j in ((2, TRN - 3), (0, TRN - 2), (1, TRN - 1)):
                pltpu.make_async_copy(
                    g_hbm.at[sb.at[j]], rows[lane], sems[lane]).wait()
                pltpu.sync_copy(rows[lane], acc.at[db.at[j]], add=True)
            if T < NTRN - 1:
                hpre.wait()

        plsc.subcore_barrier()
        pltpu.sync_copy(acc.at[pl.ds(s * RPS, RPS)],
                        out_hbm.at[c, pl.ds(s * RPS, RPS)])

    return degree_sc, propagate_sc


def _c1_body(dp_ref, x_ref, w_ref, g_ref, dcol_ref):
    dva = lax.rsqrt(dp_ref[0, 0, 0, :] + dp_ref[1, 0, 0, :] + 1.0)
    dvb = lax.rsqrt(dp_ref[0, 1, 0, :] + dp_ref[1, 1, 0, :] + 1.0)
    dv = jnp.concatenate([dva, dvb]).reshape(1, BLKH)
    rr = lax.broadcasted_iota(jnp.int32, (BLKH, BLKH), 0)
    cc = lax.broadcasted_iota(jnp.int32, (BLKH, BLKH), 1)
    diag = jnp.where(rr == cc, jnp.broadcast_to(dv, (BLKH, BLKH)), 0.0)
    dcol = jnp.dot(diag, jnp.ones((BLKH, FEAT), jnp.float32),
                   preferred_element_type=jnp.float32)
    dcol_ref[...] = dcol
    g_ref[...] = jnp.dot(x_ref[...], w_ref[...],
                         preferred_element_type=jnp.float32) * dcol


def _cmid_body(a0_ref, a1_ref, gp_ref, dcol_ref, b_ref, w_ref,
               g_ref, act_ref):
    dcol = dcol_ref[...]
    act = jnp.maximum((a0_ref[...] + a1_ref[...] + gp_ref[...]) * dcol
                      + b_ref[...], 0.0)
    act_ref[...] = act
    g_ref[...] = jnp.dot(act, w_ref[...],
                         preferred_element_type=jnp.float32) * dcol


def _pool_body(act_ref, oh_ref, pool_ref):
    @pl.when(pl.program_id(0) == 0)
    def _():
        pool_ref[...] = jnp.zeros_like(pool_ref)

    pool_ref[...] += lax.dot_general(
        oh_ref[...], act_ref[...], (((0,), (0,)), ((), ())),
        preferred_element_type=jnp.float32)


def _cfin_body(a0_ref, a1_ref, gp_ref, dcol_ref, b_ref, oh_ref, pool_ref):
    act = jnp.maximum((a0_ref[...] + a1_ref[...] + gp_ref[...]) * dcol_ref[...]
                      + b_ref[...], 0.0)

    @pl.when(pl.program_id(0) == 0)
    def _():
        pool_ref[...] = jnp.zeros_like(pool_ref)

    pool_ref[...] += lax.dot_general(
        oh_ref[...], act, (((0,), (0,)), ((), ())),
        preferred_element_type=jnp.float32)


_row_spec = pl.BlockSpec((BLKH, FEAT), lambda m: (m, 0))
_dp_spec = pl.BlockSpec((2, 2, 1, BLK), lambda m: (0, m, 0, 0))
_w_spec = pl.BlockSpec((FEAT, FEAT), lambda m: (0, 0))
_b_spec = pl.BlockSpec((1, FEAT), lambda m: (0, 0))
_oh_spec = pl.BlockSpec((BLKH, N_GRAPHS), lambda m: (m, 0))
_pool_spec = pl.BlockSpec((N_GRAPHS, FEAT), lambda m: (0, 0))


def _c1_tc(dp, x_pad, W):
    return pl.pallas_call(
        _c1_body,
        grid=(NBLKH,),
        in_specs=[_dp_spec, _row_spec, _w_spec],
        out_specs=[_row_spec, _row_spec],
        out_shape=[jax.ShapeDtypeStruct((R_PAD, FEAT), jnp.float32),
                   jax.ShapeDtypeStruct((R_PAD, FEAT), jnp.float32)],
    )(dp, x_pad, W)


def _cmid_tc(a0, a1, gp, dcol, b, W):
    return pl.pallas_call(
        _cmid_body,
        grid=(NBLKH,),
        in_specs=[_row_spec, _row_spec, _row_spec, _row_spec,
                  _b_spec, _w_spec],
        out_specs=[_row_spec, _row_spec],
        out_shape=[jax.ShapeDtypeStruct((R_PAD, FEAT), jnp.float32),
                   jax.ShapeDtypeStruct((R_PAD, FEAT), jnp.float32)],
    )(a0, a1, gp, dcol, b, W)


def _pool_tc(act, oh):
    return pl.pallas_call(
        _pool_body,
        grid=(NBLKH,),
        in_specs=[_row_spec, _oh_spec],
        out_specs=_pool_spec,
        out_shape=jax.ShapeDtypeStruct((N_GRAPHS, FEAT), jnp.float32),
    )(act, oh)


def _cfin_tc(a0, a1, gp, dcol, b, oh):
    return pl.pallas_call(
        _cfin_body,
        grid=(NBLKH,),
        in_specs=[_row_spec, _row_spec, _row_spec, _row_spec,
                  _b_spec, _oh_spec],
        out_specs=_pool_spec,
        out_shape=jax.ShapeDtypeStruct((N_GRAPHS, FEAT), jnp.float32),
    )(a0, a1, gp, dcol, b, oh)


def kernel(x, edge_index, batch, W1, b1, W2, b2, W3, b3):
    pad_n = E_PAD - N_EDGES
    src = jnp.concatenate(
        [edge_index[0].astype(jnp.int32),
         (jnp.arange(pad_n, dtype=jnp.int32) * 997) % N_NODES]
    ).reshape(E_ROWS, CHUNK)
    dst = jnp.concatenate(
        [edge_index[1].astype(jnp.int32),
         N_NODES + (jnp.arange(pad_n, dtype=jnp.int32)
                    % (R_PAD - N_NODES))]).reshape(E_ROWS, CHUNK)
    e2 = jnp.stack([src, dst])

    x_pad = jnp.pad(x, ((0, R_PAD - N_NODES), (0, 0)))
    oh = (batch[:, None] == jnp.arange(N_GRAPHS, dtype=batch.dtype)[None, :])
    oh = jnp.pad(oh.astype(jnp.float32), ((0, R_PAD - N_NODES), (0, 0)))
    b1r = b1.reshape(1, FEAT)
    b2r = b2.reshape(1, FEAT)
    b3r = b3.reshape(1, FEAT)

    degree_sc, propagate_sc = _sc_kernels()
    dp = degree_sc(dst).reshape(NC, NBLK, 1, BLK)

    g1, dcol = _c1_tc(dp, x_pad, W1)
    a1 = propagate_sc(g1, e2)
    g2, act1 = _cmid_tc(a1[0], a1[1], g1, dcol, b1r, W2)
    a2 = propagate_sc(g2, e2)
    pool1 = _pool_tc(act1, oh)
    g3, act2 = _cmid_tc(a2[0], a2[1], g2, dcol, b2r, W3)
    a3 = propagate_sc(g3, e2)
    pool2 = _pool_tc(act2, oh)
    pool3 = _cfin_tc(a3[0], a3[1], g3, dcol, b3r, oh)

    return jnp.concatenate([pool1, pool2, pool3], axis=1)


# speedup vs baseline: 3.5080x; 1.0128x over previous
import functools

import jax
import jax.numpy as jnp
from jax import lax
from jax.experimental import pallas as pl
from jax.experimental.pallas import tpu as pltpu
from jax.experimental.pallas import tpu_sc as plsc

N_NODES = 10000
N_EDGES = 320000
FEAT = 128
N_GRAPHS = 64

BLK = 128
BLKH = 256

R_PAD = 10240
NBLK = R_PAD // BLK
NBLKH = R_PAD // BLKH

NC = 2
NSUB = 16
NW = NC * NSUB
CHUNK = 80
WROWS = 128
E_PAD = NW * WROWS * CHUNK
E_ROWS = E_PAD // CHUNK
TRN = 32
NTRN = WROWS // TRN
RPS = R_PAD // NSUB

@functools.cache
def _sc_kernels():
    mesh = plsc.VectorSubcoreMesh(core_axis_name="c", subcore_axis_name="s")

    @functools.partial(
        pl.kernel,
        out_type=jax.ShapeDtypeStruct((NC * R_PAD,), jnp.float32),
        mesh=mesh,
        scratch_types=[
            pltpu.VMEM((WROWS, CHUNK), jnp.int32),
            pltpu.VMEM((CHUNK,), jnp.float32),
            pltpu.VMEM((RPS,), jnp.float32),
            pltpu.VMEM_SHARED((R_PAD,), jnp.float32),
            pltpu.SemaphoreType.DMA,
        ],
    )
    def degree_sc(dst_hbm, out_hbm, dbuf, ones_v, zbuf, dacc, sem):
        c = lax.axis_index("c")
        s = lax.axis_index("s")
        wid = c * NSUB + s

        @pl.loop(0, CHUNK // 16)
        def _(i):
            ones_v[pl.ds(i * 16, 16)] = jnp.ones((16,), jnp.float32)

        @pl.loop(0, RPS // 16)
        def _(i):
            zbuf[pl.ds(i * 16, 16)] = jnp.zeros((16,), jnp.float32)

        pltpu.sync_copy(zbuf, dacc.at[pl.ds(s * RPS, RPS)])
        pltpu.sync_copy(dst_hbm.at[pl.ds(wid * WROWS, WROWS)], dbuf)
        plsc.subcore_barrier()

        @pl.loop(0, WROWS // 4)
        def _(t):
            j = 4 * t
            h0 = pltpu.async_copy(ones_v, dacc.at[dbuf.at[j]], sem, add=True)
            h1 = pltpu.async_copy(ones_v, dacc.at[dbuf.at[j + 1]], sem, add=True)
            h2 = pltpu.async_copy(ones_v, dacc.at[dbuf.at[j + 2]], sem, add=True)
            h3 = pltpu.async_copy(ones_v, dacc.at[dbuf.at[j + 3]], sem, add=True)
            h0.wait()
            h1.wait()
            h2.wait()
            h3.wait()

        plsc.subcore_barrier()
        pltpu.sync_copy(dacc.at[pl.ds(s * RPS, RPS)],
                        out_hbm.at[pl.ds(c * R_PAD + s * RPS, RPS)])

    @functools.partial(
        pl.kernel,
        out_type=jax.ShapeDtypeStruct((NC, R_PAD, FEAT), jnp.float32),
        mesh=mesh,
        scratch_types=[
            pltpu.VMEM((2, 2, TRN, CHUNK), jnp.int32),
            pltpu.VMEM((3 * CHUNK, FEAT), jnp.float32),
            pltpu.VMEM_SHARED((R_PAD, FEAT), jnp.float32),
            pltpu.SemaphoreType.DMA,
            pltpu.SemaphoreType.DMA,
            pltpu.SemaphoreType.DMA,
            pltpu.SemaphoreType.DMA,
        ],
    )
    def propagate_sc(g_hbm, e2_hbm, out_hbm,
                     ibuf, rowsb, acc, sem0, sem1, sem2, isem):
        c = lax.axis_index("c")
        s = lax.axis_index("s")
        wid = c * NSUB + s
        rows = [rowsb.at[pl.ds(k * CHUNK, CHUNK)] for k in range(3)]
        sems = [sem0, sem1, sem2]

        @pl.loop(0, 3 * CHUNK)
        def _(i):
            @pl.loop(0, FEAT // 16)
            def _(j):
                rowsb[i, pl.ds(j * 16, 16)] = jnp.zeros((16,), jnp.float32)

        @pl.loop(0, RPS // CHUNK)
        def _(k):
            pltpu.sync_copy(rows[0], acc.at[pl.ds(s * RPS + k * CHUNK, CHUNK)])

        base = wid * WROWS
        pltpu.sync_copy(e2_hbm.at[:, pl.ds(base, TRN)], ibuf.at[0])
        plsc.subcore_barrier()

        for T in range(NTRN):
            r = T % 2
            rn = (T + 1) % 2
            if T < NTRN - 1:
                hpre = pltpu.async_copy(
                    e2_hbm.at[:, pl.ds(base + (T + 1) * TRN, TRN)],
                    ibuf.at[rn], isem)
            sb = ibuf.at[r].at[0]
            db = ibuf.at[r].at[1]
            for u in range(3):
                pltpu.async_copy(g_hbm.at[sb.at[u]], rows[u], sems[u])

            @pl.loop(0, TRN // 3 - 1)
            def _(t):
                for lane in range(3):
                    j = 3 * t + lane
                    pltpu.make_async_copy(
                        g_hbm.at[sb.at[j]], rows[lane], sems[lane]).wait()
                    pltpu.sync_copy(rows[lane], acc.at[db.at[j]], add=True)
                    pltpu.async_copy(
                        g_hbm.at[sb.at[j + 3]], rows[lane], sems[lane])

            for lane, j in ((0, TRN - 5), (1, TRN - 4)):
                pltpu.make_async_copy(
                    g_hbm.at[sb.at[j]], rows[lane], sems[lane]).wait()
                pltpu.sync_copy(rows[lane], acc.at[db.at[j]], add=True)
                pltpu.async_copy(
                    g_hbm.at[sb.at[j + 3]], rows[lane], sems[lane])
            for lane, j in ((2, TRN - 3), (0, TRN - 2), (1, TRN - 1)):
                pltpu.make_async_copy(
                    g_hbm.at[sb.at[j]], rows[lane], sems[lane]).wait()
                pltpu.sync_copy(rows[lane], acc.at[db.at[j]], add=True)
            if T < NTRN - 1:
                hpre.wait()

        plsc.subcore_barrier()
        pltpu.sync_copy(acc.at[pl.ds(s * RPS, RPS)],
                        out_hbm.at[c, pl.ds(s * RPS, RPS)])

    return degree_sc, propagate_sc


def _c1_body(dp_ref, x_ref, w_ref, bt_ref, g_ref, dcol_ref, oht_ref):
    gids = lax.broadcasted_iota(jnp.int32, (N_GRAPHS, BLKH), 0)
    oht_ref[...] = (gids == jnp.broadcast_to(bt_ref[...], (N_GRAPHS, BLKH))
                    ).astype(jnp.float32)
    dva = lax.rsqrt(dp_ref[0, 0, 0, :] + dp_ref[1, 0, 0, :] + 1.0)
    dvb = lax.rsqrt(dp_ref[0, 1, 0, :] + dp_ref[1, 1, 0, :] + 1.0)
    dv = jnp.concatenate([dva, dvb]).reshape(1, BLKH)
    rr = lax.broadcasted_iota(jnp.int32, (BLKH, BLKH), 0)
    cc = lax.broadcasted_iota(jnp.int32, (BLKH, BLKH), 1)
    diag = jnp.where(rr == cc, jnp.broadcast_to(dv, (BLKH, BLKH)), 0.0)
    dcol = jnp.dot(diag, jnp.ones((BLKH, FEAT), jnp.float32),
                   preferred_element_type=jnp.float32)
    dcol_ref[...] = dcol
    rmask = (lax.broadcasted_iota(jnp.int32, (BLKH, 1), 0)
             + pl.program_id(0) * BLKH) < N_NODES
    g_ref[...] = jnp.where(
        rmask,
        jnp.dot(x_ref[...], w_ref[...],
                preferred_element_type=jnp.float32) * dcol,
        0.0)


def _cmid_body(a0_ref, a1_ref, gp_ref, dcol_ref, b_ref, w_ref,
               g_ref, act_ref):
    dcol = dcol_ref[...]
    act = jnp.maximum((a0_ref[...] + a1_ref[...] + gp_ref[...]) * dcol
                      + b_ref[...], 0.0)
    act_ref[...] = act
    g_ref[...] = jnp.dot(act, w_ref[...],
                         preferred_element_type=jnp.float32) * dcol


def _pool_body(act_ref, oht_ref, pool_ref):
    @pl.when(pl.program_id(0) == 0)
    def _():
        pool_ref[...] = jnp.zeros_like(pool_ref)

    pool_ref[...] += lax.dot_general(
        oht_ref[...], act_ref[...], (((1,), (0,)), ((), ())),
        preferred_element_type=jnp.float32)


def _cfin_body(a0_ref, a1_ref, gp_ref, dcol_ref, b_ref, oht_ref, pool_ref):
    act = jnp.maximum((a0_ref[...] + a1_ref[...] + gp_ref[...]) * dcol_ref[...]
                      + b_ref[...], 0.0)

    @pl.when(pl.program_id(0) == 0)
    def _():
        pool_ref[...] = jnp.zeros_like(pool_ref)

    pool_ref[...] += lax.dot_general(
        oht_ref[...], act, (((1,), (0,)), ((), ())),
        preferred_element_type=jnp.float32)


_row_spec = pl.BlockSpec((BLKH, FEAT), lambda m: (m, 0))
_dp_spec = pl.BlockSpec((2, 2, 1, BLK), lambda m: (0, m, 0, 0))
_w_spec = pl.BlockSpec((FEAT, FEAT), lambda m: (0, 0))
_b_spec = pl.BlockSpec((1, FEAT), lambda m: (0, 0))
_oht_spec = pl.BlockSpec((N_GRAPHS, BLKH), lambda m: (0, m))
_bt_spec = pl.BlockSpec((1, BLKH), lambda m: (0, m))
_pool_spec = pl.BlockSpec((N_GRAPHS, FEAT), lambda m: (0, 0))


def _c1_tc(dp, x, W, bt):
    return pl.pallas_call(
        _c1_body,
        grid=(NBLKH,),
        in_specs=[_dp_spec, _row_spec, _w_spec, _bt_spec],
        out_specs=[_row_spec, _row_spec, _oht_spec],
        out_shape=[jax.ShapeDtypeStruct((R_PAD, FEAT), jnp.float32),
                   jax.ShapeDtypeStruct((R_PAD, FEAT), jnp.float32),
                   jax.ShapeDtypeStruct((N_GRAPHS, R_PAD), jnp.float32)],
    )(dp, x, W, bt)


def _cmid_tc(a0, a1, gp, dcol, b, W):
    return pl.pallas_call(
        _cmid_body,
        grid=(NBLKH,),
        in_specs=[_row_spec, _row_spec, _row_spec, _row_spec,
                  _b_spec, _w_spec],
        out_specs=[_row_spec, _row_spec],
        out_shape=[jax.ShapeDtypeStruct((R_PAD, FEAT), jnp.float32),
                   jax.ShapeDtypeStruct((R_PAD, FEAT), jnp.float32)],
    )(a0, a1, gp, dcol, b, W)


def _pool_tc(act, oht):
    return pl.pallas_call(
        _pool_body,
        grid=(NBLKH,),
        in_specs=[_row_spec, _oht_spec],
        out_specs=_pool_spec,
        out_shape=jax.ShapeDtypeStruct((N_GRAPHS, FEAT), jnp.float32),
    )(act, oht)


def _cfin_tc(a0, a1, gp, dcol, b, oht):
    return pl.pallas_call(
        _cfin_body,
        grid=(NBLKH,),
        in_specs=[_row_spec, _row_spec, _row_spec, _row_spec,
                  _b_spec, _oht_spec],
        out_specs=_pool_spec,
        out_shape=jax.ShapeDtypeStruct((N_GRAPHS, FEAT), jnp.float32),
    )(a0, a1, gp, dcol, b, oht)


def kernel(x, edge_index, batch, W1, b1, W2, b2, W3, b3):
    pad_n = E_PAD - N_EDGES
    src = jnp.concatenate(
        [edge_index[0].astype(jnp.int32),
         (jnp.arange(pad_n, dtype=jnp.int32) * 997) % N_NODES]
    ).reshape(E_ROWS, CHUNK)
    dst = jnp.concatenate(
        [edge_index[1].astype(jnp.int32),
         N_NODES + (jnp.arange(pad_n, dtype=jnp.int32)
                    % (R_PAD - N_NODES))]).reshape(E_ROWS, CHUNK)
    e2 = jnp.stack([src, dst])

    bt = jnp.pad(batch.astype(jnp.int32), (0, R_PAD - N_NODES),
                 constant_values=N_GRAPHS).reshape(1, R_PAD)
    b1r = b1.reshape(1, FEAT)
    b2r = b2.reshape(1, FEAT)
    b3r = b3.reshape(1, FEAT)

    degree_sc, propagate_sc = _sc_kernels()
    dp = degree_sc(dst).reshape(NC, NBLK, 1, BLK)

    g1, dcol, oht = _c1_tc(dp, x, W1, bt)
    a1 = propagate_sc(g1, e2)
    g2, act1 = _cmid_tc(a1[0], a1[1], g1, dcol, b1r, W2)
    a2 = propagate_sc(g2, e2)
    pool1 = _pool_tc(act1, oht)
    g3, act2 = _cmid_tc(a2[0], a2[1], g2, dcol, b2r, W3)
    a3 = propagate_sc(g3, e2)
    pool2 = _pool_tc(act2, oht)
    pool3 = _cfin_tc(a3[0], a3[1], g3, dcol, b3r, oht)

    return jnp.concatenate([pool1, pool2, pool3], axis=1)


# speedup vs baseline: 4.0111x; 1.1434x over previous
import functools

import jax
import jax.numpy as jnp
from jax import lax
from jax.experimental import pallas as pl
from jax.experimental.pallas import tpu as pltpu
from jax.experimental.pallas import tpu_sc as plsc

N_NODES = 10000
N_EDGES = 320000
FEAT = 128
N_GRAPHS = 64

BLK = 128
BLKH = 512

R_PAD = 10240
NBLK = R_PAD // BLK
NBLKH = R_PAD // BLKH

NC = 2
NSUB = 16
NW = NC * NSUB
CHUNK = 80
WROWS = 128
E_PAD = NW * WROWS * CHUNK
E_ROWS = E_PAD // CHUNK
TRN = 32
NTRN = WROWS // TRN
RPS = R_PAD // NSUB

@functools.cache
def _sc_kernels():
    mesh = plsc.VectorSubcoreMesh(core_axis_name="c", subcore_axis_name="s")

    @functools.partial(
        pl.kernel,
        out_type=jax.ShapeDtypeStruct((NC * R_PAD,), jnp.float32),
        mesh=mesh,
        scratch_types=[
            pltpu.VMEM((WROWS, CHUNK), jnp.int32),
            pltpu.VMEM((CHUNK,), jnp.float32),
            pltpu.VMEM((RPS,), jnp.float32),
            pltpu.VMEM_SHARED((R_PAD,), jnp.float32),
            pltpu.SemaphoreType.DMA,
        ],
    )
    def degree_sc(dst_hbm, out_hbm, dbuf, ones_v, zbuf, dacc, sem):
        c = lax.axis_index("c")
        s = lax.axis_index("s")
        wid = c * NSUB + s

        @pl.loop(0, CHUNK // 16)
        def _(i):
            ones_v[pl.ds(i * 16, 16)] = jnp.ones((16,), jnp.float32)

        @pl.loop(0, RPS // 16)
        def _(i):
            zbuf[pl.ds(i * 16, 16)] = jnp.zeros((16,), jnp.float32)

        pltpu.sync_copy(zbuf, dacc.at[pl.ds(s * RPS, RPS)])
        pltpu.sync_copy(dst_hbm.at[pl.ds(wid * WROWS, WROWS)], dbuf)
        plsc.subcore_barrier()

        @pl.loop(0, WROWS // 4)
        def _(t):
            j = 4 * t
            h0 = pltpu.async_copy(ones_v, dacc.at[dbuf.at[j]], sem, add=True)
            h1 = pltpu.async_copy(ones_v, dacc.at[dbuf.at[j + 1]], sem, add=True)
            h2 = pltpu.async_copy(ones_v, dacc.at[dbuf.at[j + 2]], sem, add=True)
            h3 = pltpu.async_copy(ones_v, dacc.at[dbuf.at[j + 3]], sem, add=True)
            h0.wait()
            h1.wait()
            h2.wait()
            h3.wait()

        plsc.subcore_barrier()
        pltpu.sync_copy(dacc.at[pl.ds(s * RPS, RPS)],
                        out_hbm.at[pl.ds(c * R_PAD + s * RPS, RPS)])

    @functools.partial(
        pl.kernel,
        out_type=jax.ShapeDtypeStruct((NC, R_PAD, FEAT), jnp.float32),
        mesh=mesh,
        scratch_types=[
            pltpu.VMEM((2, 2, TRN, CHUNK), jnp.int32),
            pltpu.VMEM((3 * CHUNK, FEAT), jnp.float32),
            pltpu.VMEM_SHARED((R_PAD, FEAT), jnp.float32),
            pltpu.SemaphoreType.DMA,
            pltpu.SemaphoreType.DMA,
            pltpu.SemaphoreType.DMA,
            pltpu.SemaphoreType.DMA,
        ],
    )
    def propagate_sc(g_hbm, e2_hbm, out_hbm,
                     ibuf, rowsb, acc, sem0, sem1, sem2, isem):
        c = lax.axis_index("c")
        s = lax.axis_index("s")
        wid = c * NSUB + s
        rows = [rowsb.at[pl.ds(k * CHUNK, CHUNK)] for k in range(3)]
        sems = [sem0, sem1, sem2]

        @pl.loop(0, 3 * CHUNK)
        def _(i):
            @pl.loop(0, FEAT // 16)
            def _(j):
                rowsb[i, pl.ds(j * 16, 16)] = jnp.zeros((16,), jnp.float32)

        @pl.loop(0, RPS // CHUNK)
        def _(k):
            pltpu.sync_copy(rows[0], acc.at[pl.ds(s * RPS + k * CHUNK, CHUNK)])

        base = wid * WROWS
        pltpu.sync_copy(e2_hbm.at[:, pl.ds(base, TRN)], ibuf.at[0])
        plsc.subcore_barrier()

        for T in range(NTRN):
            r = T % 2
            rn = (T + 1) % 2
            if T < NTRN - 1:
                hpre = pltpu.async_copy(
                    e2_hbm.at[:, pl.ds(base + (T + 1) * TRN, TRN)],
                    ibuf.at[rn], isem)
            sb = ibuf.at[r].at[0]
            db = ibuf.at[r].at[1]
            for u in range(3):
                pltpu.async_copy(g_hbm.at[sb.at[u]], rows[u], sems[u])

            @pl.loop(0, TRN // 3 - 1)
            def _(t):
                for lane in range(3):
                    j = 3 * t + lane
                    pltpu.make_async_copy(
                        g_hbm.at[sb.at[j]], rows[lane], sems[lane]).wait()
                    pltpu.sync_copy(rows[lane], acc.at[db.at[j]], add=True)
                    pltpu.async_copy(
                        g_hbm.at[sb.at[j + 3]], rows[lane], sems[lane])

            for lane, j in ((0, TRN - 5), (1, TRN - 4)):
                pltpu.make_async_copy(
                    g_hbm.at[sb.at[j]], rows[lane], sems[lane]).wait()
                pltpu.sync_copy(rows[lane], acc.at[db.at[j]], add=True)
                pltpu.async_copy(
                    g_hbm.at[sb.at[j + 3]], rows[lane], sems[lane])
            for lane, j in ((2, TRN - 3), (0, TRN - 2), (1, TRN - 1)):
                pltpu.make_async_copy(
                    g_hbm.at[sb.at[j]], rows[lane], sems[lane]).wait()
                pltpu.sync_copy(rows[lane], acc.at[db.at[j]], add=True)
            if T < NTRN - 1:
                hpre.wait()

        plsc.subcore_barrier()
        pltpu.sync_copy(acc.at[pl.ds(s * RPS, RPS)],
                        out_hbm.at[c, pl.ds(s * RPS, RPS)])

    return degree_sc, propagate_sc


def _c1_body(dp_ref, x_ref, w_ref, bt_ref, g_ref, dcol_ref, oht_ref):
    gids = lax.broadcasted_iota(jnp.int32, (N_GRAPHS, BLKH), 0)
    oht_ref[...] = (gids == jnp.broadcast_to(bt_ref[...], (N_GRAPHS, BLKH))
                    ).astype(jnp.float32)
    dv = jnp.concatenate(
        [lax.rsqrt(dp_ref[0, q, 0, :] + dp_ref[1, q, 0, :] + 1.0)
         for q in range(BLKH // BLK)]).reshape(1, BLKH)
    rr = lax.broadcasted_iota(jnp.int32, (BLKH, BLKH), 0)
    cc = lax.broadcasted_iota(jnp.int32, (BLKH, BLKH), 1)
    diag = jnp.where(rr == cc, jnp.broadcast_to(dv, (BLKH, BLKH)), 0.0)
    dcol = jnp.dot(diag, jnp.ones((BLKH, FEAT), jnp.float32),
                   preferred_element_type=jnp.float32)
    dcol_ref[...] = dcol
    rmask = (lax.broadcasted_iota(jnp.int32, (BLKH, 1), 0)
             + pl.program_id(0) * BLKH) < N_NODES
    g_ref[...] = jnp.where(
        rmask,
        jnp.dot(x_ref[...], w_ref[...],
                preferred_element_type=jnp.float32) * dcol,
        0.0)


def _cmid_body(a0_ref, a1_ref, gp_ref, dcol_ref, b_ref, w_ref,
               g_ref, act_ref):
    dcol = dcol_ref[...]
    act = jnp.maximum((a0_ref[0] + a1_ref[0] + gp_ref[...]) * dcol
                      + b_ref[...], 0.0)
    act_ref[...] = act
    g_ref[...] = jnp.dot(act, w_ref[...],
                         preferred_element_type=jnp.float32) * dcol


def _pool_body(act_ref, oht_ref, pool_ref):
    @pl.when(pl.program_id(0) == 0)
    def _():
        pool_ref[...] = jnp.zeros_like(pool_ref)

    pool_ref[...] += lax.dot_general(
        oht_ref[...], act_ref[...], (((1,), (0,)), ((), ())),
        preferred_element_type=jnp.float32)


def _cfin_body(a0_ref, a1_ref, gp_ref, dcol_ref, b_ref, oht_ref, pool_ref):
    act = jnp.maximum((a0_ref[0] + a1_ref[0] + gp_ref[...]) * dcol_ref[...]
                      + b_ref[...], 0.0)

    @pl.when(pl.program_id(0) == 0)
    def _():
        pool_ref[...] = jnp.zeros_like(pool_ref)

    pool_ref[...] += lax.dot_general(
        oht_ref[...], act, (((1,), (0,)), ((), ())),
        preferred_element_type=jnp.float32)


_row_spec = pl.BlockSpec((BLKH, FEAT), lambda m: (m, 0))
_dp_spec = pl.BlockSpec((2, BLKH // BLK, 1, BLK), lambda m: (0, m, 0, 0))
_w_spec = pl.BlockSpec((FEAT, FEAT), lambda m: (0, 0))
_b_spec = pl.BlockSpec((1, FEAT), lambda m: (0, 0))
_oht_spec = pl.BlockSpec((N_GRAPHS, BLKH), lambda m: (0, m))
_bt_spec = pl.BlockSpec((1, BLKH), lambda m: (0, m))
_a0_spec = pl.BlockSpec((1, BLKH, FEAT), lambda m: (0, m, 0))
_a1_spec = pl.BlockSpec((1, BLKH, FEAT), lambda m: (1, m, 0))
_pool_spec = pl.BlockSpec((N_GRAPHS, FEAT), lambda m: (0, 0))


def _c1_tc(dp, x, W, bt):
    return pl.pallas_call(
        _c1_body,
        grid=(NBLKH,),
        in_specs=[_dp_spec, _row_spec, _w_spec, _bt_spec],
        out_specs=[_row_spec, _row_spec, _oht_spec],
        out_shape=[jax.ShapeDtypeStruct((R_PAD, FEAT), jnp.float32),
                   jax.ShapeDtypeStruct((R_PAD, FEAT), jnp.float32),
                   jax.ShapeDtypeStruct((N_GRAPHS, R_PAD), jnp.float32)],
    )(dp, x, W, bt)


def _cmid_tc(a, gp, dcol, b, W):
    return pl.pallas_call(
        _cmid_body,
        grid=(NBLKH,),
        in_specs=[_a0_spec, _a1_spec, _row_spec, _row_spec,
                  _b_spec, _w_spec],
        out_specs=[_row_spec, _row_spec],
        out_shape=[jax.ShapeDtypeStruct((R_PAD, FEAT), jnp.float32),
                   jax.ShapeDtypeStruct((R_PAD, FEAT), jnp.float32)],
    )(a, a, gp, dcol, b, W)


def _pool_tc(act, oht):
    return pl.pallas_call(
        _pool_body,
        grid=(NBLKH,),
        in_specs=[_row_spec, _oht_spec],
        out_specs=_pool_spec,
        out_shape=jax.ShapeDtypeStruct((N_GRAPHS, FEAT), jnp.float32),
    )(act, oht)


def _cfin_tc(a, gp, dcol, b, oht):
    return pl.pallas_call(
        _cfin_body,
        grid=(NBLKH,),
        in_specs=[_a0_spec, _a1_spec, _row_spec, _row_spec,
                  _b_spec, _oht_spec],
        out_specs=_pool_spec,
        out_shape=jax.ShapeDtypeStruct((N_GRAPHS, FEAT), jnp.float32),
    )(a, a, gp, dcol, b, oht)


def kernel(x, edge_index, batch, W1, b1, W2, b2, W3, b3):
    pad_n = E_PAD - N_EDGES
    src = jnp.concatenate(
        [edge_index[0].astype(jnp.int32),
         (jnp.arange(pad_n, dtype=jnp.int32) * 997) % N_NODES]
    ).reshape(E_ROWS, CHUNK)
    dst = jnp.concatenate(
        [edge_index[1].astype(jnp.int32),
         N_NODES + (jnp.arange(pad_n, dtype=jnp.int32)
                    % (R_PAD - N_NODES))]).reshape(E_ROWS, CHUNK)
    e2 = jnp.stack([src, dst])

    bt = jnp.pad(batch.astype(jnp.int32), (0, R_PAD - N_NODES),
                 constant_values=N_GRAPHS).reshape(1, R_PAD)
    b1r = b1.reshape(1, FEAT)
    b2r = b2.reshape(1, FEAT)
    b3r = b3.reshape(1, FEAT)

    degree_sc, propagate_sc = _sc_kernels()
    dp = degree_sc(dst).reshape(NC, NBLK, 1, BLK)

    g1, dcol, oht = _c1_tc(dp, x, W1, bt)
    a1 = propagate_sc(g1, e2)
    g2, act1 = _cmid_tc(a1, g1, dcol, b1r, W2)
    a2 = propagate_sc(g2, e2)
    pool1 = _pool_tc(act1, oht)
    g3, act2 = _cmid_tc(a2, g2, dcol, b2r, W3)
    a3 = propagate_sc(g3, e2)
    pool2 = _pool_tc(act2, oht)
    pool3 = _cfin_tc(a3, g3, dcol, b3r, oht)

    return jnp.concatenate([pool1, pool2, pool3], axis=1)
